# baseline (speedup 1.0000x reference)
import functools

import jax
import jax.numpy as jnp
from jax.experimental import pallas as pl
from jax.experimental.pallas import tpu as pltpu

HIGH = jax.lax.Precision.HIGHEST
FAST = jax.lax.Precision.DEFAULT


def _dot(a, b, trans_a=False, trans_b=False, precision=HIGH):
    ca = 0 if trans_a else a.ndim - 1
    cb = 1 if trans_b else 0
    return jax.lax.dot_general(a, b, (((ca,), (cb,)), ((), ())),
                               precision=precision,
                               preferred_element_type=jnp.float32)


def _stats_kernel(x_ref, wf_ref, bf_ref, wg_ref, bg_ref, be_ref,
                  a_ref, gate_ref, coeff_ref, gbe_ref):
    xm = jnp.mean(x_ref[...], axis=1)
    adapt = _dot(xm, wf_ref[...], trans_b=True) + bf_ref[...]
    a = jnp.mean(adapt, axis=0, keepdims=True)
    logits = _dot(adapt, wg_ref[...], trans_b=True) + bg_ref[...]
    m = jnp.max(logits, axis=-1, keepdims=True)
    p = jnp.exp(logits - m)
    gate = p / jnp.sum(p, axis=-1, keepdims=True)
    Bb = gate.shape[0]
    coeff = (gate[:, :, None] * a[0][None, None, :]).reshape(Bb, -1)
    a_ref[...] = a
    gate_ref[...] = gate
    coeff_ref[...] = coeff
    gbe_ref[...] = _dot(gate, be_ref[...])


def _mega_kernel(a_ref, coeff_ref, gate_ref,
                 awt_ref, wt_ref, abt_ref, awc_ref, wc_ref, abc_ref,
                 awe0_ref, awe1_ref, we_ref, abe_ref,
                 wo_ref, btm_ref, bcm_ref, gbe_ref, bo_ref, x_ref,
                 out_ref,
                 amat_s, bmat_s, wmix_s, v_s, *, rows, d, ns, nsb):
    i = pl.program_id(0)
    b = gate_ref.shape[0]
    e = gate_ref.shape[1]

    @pl.when(i < ns)
    def _stream():
        sl = pl.ds(i * rows, rows)
        dwa = _dot(a_ref[...], awt_ref[...], precision=FAST)
        amat_s[sl, :] = wt_ref[...] + dwa.reshape(rows, d) + abt_ref[...]
        dwb = _dot(a_ref[...], awc_ref[...], precision=FAST)
        bmat_s[sl, :] = wc_ref[...] + dwb.reshape(rows, d) + abc_ref[...]
        half = awe0_ref.shape[0]
        dw = (_dot(coeff_ref[:, :half], awe0_ref[...], precision=FAST)
              + _dot(coeff_ref[:, half:], awe1_ref[...], precision=FAST))
        gabe = _dot(gate_ref[...], abe_ref[...])
        gwe = _dot(gate_ref[...], we_ref[...].reshape(e, rows * d))
        wmix_s[:, sl, :] = (dw + gabe + gwe).reshape(b, rows, d)

    @pl.when(i == ns - 1)
    def _compose():
        t = _dot(bmat_s[...], amat_s[...], precision=FAST)
        bias2 = _dot(btm_ref[...], bmat_s[...], trans_b=True) + bcm_ref[...]
        for bb in range(b):
            wm = wmix_s[bb]
            m = _dot(wo_ref[...], _dot(wm, t, precision=FAST),
                     precision=FAST)
            h3b = _dot(bias2, wm, trans_b=True) + gbe_ref[bb:bb + 1]
            v_s[bb, :, :] = _dot(h3b, wo_ref[...], trans_b=True) + bo_ref[...]
            wmix_s[bb] = m

    @pl.when(i >= ns)
    def _apply():
        bb = (i - ns) // nsb
        m = wmix_s[pl.ds(bb, 1)][0]
        out_ref[0] = (_dot(x_ref[0], m, trans_b=True, precision=FAST)
                      + v_s[pl.ds(bb, 1)][0])


@jax.jit
def kernel(x, Wf, bf, W_tm, b_tm, AW_tm, Ab_tm, W_cm, b_cm, AW_cm, Ab_cm,
           Wg, bg, We, be, AWe, Abe, Wo, bo):
    B, S, D = x.shape
    A = Wf.shape[0]
    E = Wg.shape[0]
    f32 = jnp.float32

    a, gate, coeff, gbe = pl.pallas_call(
        _stats_kernel,
        out_shape=[
            jax.ShapeDtypeStruct((1, A), f32),
            jax.ShapeDtypeStruct((B, E), f32),
            jax.ShapeDtypeStruct((B, E * A), f32),
            jax.ShapeDtypeStruct((B, D), f32),
        ],
    )(x, Wf, bf.reshape(1, A), Wg, bg.reshape(1, E), be)

    R2 = 16
    NS = D // R2
    SB = 512
    NSB = S // SB
    grid2 = (NS + B * NSB,)

    def _col(i):
        return jnp.minimum(i, NS - 1)

    def _xb(i):
        k = jnp.maximum(i - NS, 0)
        return (k // NSB, k % NSB, 0)

    out = pl.pallas_call(
        functools.partial(_mega_kernel, rows=R2, d=D, ns=NS, nsb=NSB),
        grid=grid2,
        compiler_params=pltpu.CompilerParams(
            dimension_semantics=("arbitrary",)),
        in_specs=[
            pl.BlockSpec((1, A), lambda i: (0, 0)),
            pl.BlockSpec((B, E * A), lambda i: (0, 0)),
            pl.BlockSpec((B, E), lambda i: (0, 0)),
            pl.BlockSpec((A, R2 * D), lambda i: (0, _col(i))),
            pl.BlockSpec((R2, D), lambda i: (_col(i), 0)),
            pl.BlockSpec((R2, D), lambda i: (_col(i), 0)),
            pl.BlockSpec((A, R2 * D), lambda i: (0, _col(i))),
            pl.BlockSpec((R2, D), lambda i: (_col(i), 0)),
            pl.BlockSpec((R2, D), lambda i: (_col(i), 0)),
            pl.BlockSpec((E * A // 2, R2 * D), lambda i: (0, _col(i))),
            pl.BlockSpec((E * A // 2, R2 * D), lambda i: (1, _col(i))),
            pl.BlockSpec((E, R2, D), lambda i: (0, _col(i), 0)),
            pl.BlockSpec((E, R2 * D), lambda i: (0, _col(i))),
            pl.BlockSpec((D, D), lambda i: (0, 0)),
            pl.BlockSpec((1, D), lambda i: (0, 0)),
            pl.BlockSpec((1, D), lambda i: (0, 0)),
            pl.BlockSpec((B, D), lambda i: (0, 0)),
            pl.BlockSpec((1, D), lambda i: (0, 0)),
            pl.BlockSpec((1, SB, D), _xb),
        ],
        out_specs=pl.BlockSpec((1, SB, D), _xb),
        out_shape=jax.ShapeDtypeStruct((B, S, D), f32),
        scratch_shapes=[
            pltpu.VMEM((D, D), f32),
            pltpu.VMEM((D, D), f32),
            pltpu.VMEM((B, D, D), f32),
            pltpu.VMEM((B, 1, D), f32),
        ],
    )(a, coeff, gate,
      AW_tm, W_tm, Ab_tm.reshape(D, D), AW_cm, W_cm, Ab_cm.reshape(D, D),
      AWe.reshape(E * A, D * D), AWe.reshape(E * A, D * D), We, Abe,
      Wo, b_tm.reshape(1, D), b_cm.reshape(1, D), gbe, bo.reshape(1, D), x)
    return out


# speedup vs baseline: 1.0012x; 1.0012x over previous
import functools

import jax
import jax.numpy as jnp
from jax.experimental import pallas as pl
from jax.experimental.pallas import tpu as pltpu

HIGH = jax.lax.Precision.HIGHEST
FAST = jax.lax.Precision.DEFAULT


def _dot(a, b, trans_a=False, trans_b=False, precision=HIGH):
    ca = 0 if trans_a else a.ndim - 1
    cb = 1 if trans_b else 0
    return jax.lax.dot_general(a, b, (((ca,), (cb,)), ((), ())),
                               precision=precision,
                               preferred_element_type=jnp.float32)


def _stats_kernel(x_ref, wf_ref, bf_ref, wg_ref, bg_ref, be_ref,
                  a_ref, gate_ref, coeff_ref, gbe_ref):
    xm = jnp.mean(x_ref[...], axis=1)
    adapt = _dot(xm, wf_ref[...], trans_b=True) + bf_ref[...]
    a = jnp.mean(adapt, axis=0, keepdims=True)
    logits = _dot(adapt, wg_ref[...], trans_b=True) + bg_ref[...]
    m = jnp.max(logits, axis=-1, keepdims=True)
    p = jnp.exp(logits - m)
    gate = p / jnp.sum(p, axis=-1, keepdims=True)
    Bb = gate.shape[0]
    coeff = (gate[:, :, None] * a[0][None, None, :]).reshape(Bb, -1)
    a_ref[...] = a
    gate_ref[...] = gate
    coeff_ref[...] = coeff
    gbe_ref[...] = _dot(gate, be_ref[...])


def _mega_kernel(a_ref, coeff_ref, gate_ref,
                 awt_ref, wt_ref, abt_ref, awc_ref, wc_ref, abc_ref,
                 awe_ref, we_ref, abe_ref,
                 wo_ref, btm_ref, bcm_ref, gbe_ref, bo_ref, x_ref,
                 out_ref,
                 amat_s, bmat_s, wmix_s, v_s, *, rows, d, ns, nsb):
    i = pl.program_id(0)
    b = gate_ref.shape[0]
    e = gate_ref.shape[1]

    @pl.when(i < ns)
    def _stream():
        sl = pl.ds(i * rows, rows)
        dwa = _dot(a_ref[...], awt_ref[...], precision=FAST)
        amat_s[sl, :] = wt_ref[...] + dwa.reshape(rows, d) + abt_ref[...]
        dwb = _dot(a_ref[...], awc_ref[...], precision=FAST)
        bmat_s[sl, :] = wc_ref[...] + dwb.reshape(rows, d) + abc_ref[...]
        dw = _dot(coeff_ref[...], awe_ref[...], precision=FAST)
        gabe = _dot(gate_ref[...], abe_ref[...])
        gwe = _dot(gate_ref[...], we_ref[...].reshape(e, rows * d))
        wmix_s[:, sl, :] = (dw + gabe + gwe).reshape(b, rows, d)

    @pl.when(i == ns - 1)
    def _compose():
        t = _dot(bmat_s[...], amat_s[...], precision=FAST)
        bias2 = _dot(btm_ref[...], bmat_s[...], trans_b=True) + bcm_ref[...]
        for bb in range(b):
            wm = wmix_s[bb]
            m = _dot(wo_ref[...], _dot(wm, t, precision=FAST),
                     precision=FAST)
            h3b = _dot(bias2, wm, trans_b=True) + gbe_ref[bb:bb + 1]
            v_s[bb, :, :] = _dot(h3b, wo_ref[...], trans_b=True) + bo_ref[...]
            wmix_s[bb] = m

    @pl.when(i >= ns)
    def _apply():
        bb = (i - ns) // nsb
        m = wmix_s[pl.ds(bb, 1)][0]
        out_ref[0] = (_dot(x_ref[0], m, trans_b=True, precision=FAST)
                      + v_s[pl.ds(bb, 1)][0])


@jax.jit
def kernel(x, Wf, bf, W_tm, b_tm, AW_tm, Ab_tm, W_cm, b_cm, AW_cm, Ab_cm,
           Wg, bg, We, be, AWe, Abe, Wo, bo):
    B, S, D = x.shape
    A = Wf.shape[0]
    E = Wg.shape[0]
    f32 = jnp.float32

    a, gate, coeff, gbe = pl.pallas_call(
        _stats_kernel,
        out_shape=[
            jax.ShapeDtypeStruct((1, A), f32),
            jax.ShapeDtypeStruct((B, E), f32),
            jax.ShapeDtypeStruct((B, E * A), f32),
            jax.ShapeDtypeStruct((B, D), f32),
        ],
    )(x, Wf, bf.reshape(1, A), Wg, bg.reshape(1, E), be)

    R2 = 16
    NS = D // R2
    SB = 512
    NSB = S // SB
    grid2 = (NS + B * NSB,)

    def _col(i):
        return jnp.minimum(i, NS - 1)

    def _xb(i):
        k = jnp.maximum(i - NS, 0)
        return (k // NSB, k % NSB, 0)

    out = pl.pallas_call(
        functools.partial(_mega_kernel, rows=R2, d=D, ns=NS, nsb=NSB),
        grid=grid2,
        compiler_params=pltpu.CompilerParams(
            dimension_semantics=("arbitrary",)),
        in_specs=[
            pl.BlockSpec((1, A), lambda i: (0, 0)),
            pl.BlockSpec((B, E * A), lambda i: (0, 0)),
            pl.BlockSpec((B, E), lambda i: (0, 0)),
            pl.BlockSpec((A, R2 * D), lambda i: (0, _col(i))),
            pl.BlockSpec((R2, D), lambda i: (_col(i), 0)),
            pl.BlockSpec((R2, D), lambda i: (_col(i), 0)),
            pl.BlockSpec((A, R2 * D), lambda i: (0, _col(i))),
            pl.BlockSpec((R2, D), lambda i: (_col(i), 0)),
            pl.BlockSpec((R2, D), lambda i: (_col(i), 0)),
            pl.BlockSpec((E * A, R2 * D), lambda i: (0, _col(i))),
            pl.BlockSpec((E, R2, D), lambda i: (0, _col(i), 0)),
            pl.BlockSpec((E, R2 * D), lambda i: (0, _col(i))),
            pl.BlockSpec((D, D), lambda i: (0, 0)),
            pl.BlockSpec((1, D), lambda i: (0, 0)),
            pl.BlockSpec((1, D), lambda i: (0, 0)),
            pl.BlockSpec((B, D), lambda i: (0, 0)),
            pl.BlockSpec((1, D), lambda i: (0, 0)),
            pl.BlockSpec((1, SB, D), _xb),
        ],
        out_specs=pl.BlockSpec((1, SB, D), _xb),
        out_shape=jax.ShapeDtypeStruct((B, S, D), f32),
        scratch_shapes=[
            pltpu.VMEM((D, D), f32),
            pltpu.VMEM((D, D), f32),
            pltpu.VMEM((B, D, D), f32),
            pltpu.VMEM((B, 1, D), f32),
        ],
    )(a, coeff, gate,
      AW_tm, W_tm, Ab_tm.reshape(D, D), AW_cm, W_cm, Ab_cm.reshape(D, D),
      AWe.reshape(E * A, D * D), We, Abe,
      Wo, b_tm.reshape(1, D), b_cm.reshape(1, D), gbe, bo.reshape(1, D), x)
    return out
